# baseline (speedup 1.0000x reference)
import functools

import jax
import jax.numpy as jnp
from jax import lax
from jax.experimental import pallas as pl
from jax.experimental.pallas import tpu as pltpu
from jax.experimental.pallas import tpu_sc as plsc

TOPK = 2
BM = 128
NC = 2
NS = 16
NW = NC * NS


def _router_body(x_ref, cond_ref, wr_ref, br_ref, wc_ref,
                 topi_ref, nw_ref, aux_ref):
    T = x_ref.shape[0]
    E = wr_ref.shape[1]
    x = x_ref[...]
    logits = jnp.dot(x, wr_ref[...], preferred_element_type=jnp.float32)
    cl = jnp.dot(cond_ref[...], wc_ref[...], preferred_element_type=jnp.float32)
    logits = logits + cl + br_ref[...]
    m = jnp.max(logits, axis=-1, keepdims=True)
    p = jnp.exp(logits - m)
    p = p / jnp.sum(p, axis=-1, keepdims=True)

    iota = lax.broadcasted_iota(jnp.int32, (T, E), 1)
    m1 = jnp.max(p, axis=-1, keepdims=True)
    i1 = jnp.min(jnp.where(p >= m1, iota, E), axis=-1, keepdims=True)
    pm = jnp.where(iota == i1, -jnp.inf, p)
    m2 = jnp.max(pm, axis=-1, keepdims=True)
    i2 = jnp.min(jnp.where(pm >= m2, iota, E), axis=-1, keepdims=True)

    s = jnp.clip(m1 + m2, 1e-6, None)
    w1 = m1 / s
    w2 = m2 / s
    topi_ref[...] = jnp.concatenate([i1, i2], axis=1)
    nw_ref[...] = jnp.concatenate([w1, w2], axis=1)

    importance = jnp.sum(p, axis=0) / T
    load = jnp.sum(jnp.where(iota == i1, w1, 0.0)
                   + jnp.where(iota == i2, w2, 0.0), axis=0)
    importance = importance / jnp.clip(jnp.sum(importance), 1e-6, None)
    load = load / jnp.clip(jnp.sum(load), 1e-6, None)
    uni = 1.0 / E
    aux = jnp.mean((importance - uni) ** 2) + jnp.mean((load - uni) ** 2)
    aux_ref[...] = aux.reshape(1, 1)


def _route(x, cond, Wr, br, Wc):
    T = x.shape[0]
    E = Wr.shape[1]
    return pl.pallas_call(
        _router_body,
        out_shape=(
            jax.ShapeDtypeStruct((T, TOPK), jnp.int32),
            jax.ShapeDtypeStruct((T, TOPK), jnp.float32),
            jax.ShapeDtypeStruct((1, 1), jnp.float32),
        ),
    )(x, cond, Wr, br.reshape(1, E), Wc)


def _sc_combine(y, slot0, slot1):
    T = slot0.shape[0]
    D = y.shape[1]
    tpw = T // NW
    mesh = plsc.VectorSubcoreMesh(core_axis_name="c", subcore_axis_name="s")

    @functools.partial(
        pl.kernel, mesh=mesh,
        out_type=jax.ShapeDtypeStruct((T, D), jnp.float32),
        scratch_types=[
            pltpu.VMEM((tpw,), jnp.int32),
            pltpu.VMEM((tpw,), jnp.int32),
            pltpu.VMEM((tpw, D), jnp.float32),
            pltpu.VMEM((tpw, D), jnp.float32),
            pltpu.SemaphoreType.DMA,
        ],
    )
    def k(s0_hbm, s1_hbm, y_hbm, out_hbm, i0_v, i1_v, a_v, b_v, sem):
        wid = lax.axis_index("s") * NC + lax.axis_index("c")
        base = wid * tpw
        pltpu.sync_copy(s0_hbm.at[pl.ds(base, tpw)], i0_v)
        pltpu.sync_copy(s1_hbm.at[pl.ds(base, tpw)], i1_v)
        pltpu.async_copy(y_hbm.at[i0_v], a_v, sem).wait()
        pltpu.async_copy(y_hbm.at[i1_v], b_v, sem).wait()

        def body(r, carry):
            for c in range(D // 16):
                sl = pl.ds(c * 16, 16)
                a_v[r, sl] = a_v[r, sl] + b_v[r, sl]
            return carry

        lax.fori_loop(0, tpw, body, 0)
        pltpu.sync_copy(a_v, out_hbm.at[pl.ds(base, tpw)])

    return k(slot0, slot1, y)


def _ffn_body(be_ref, tok_ref, x_ref, ws_ref, w1_ref, b1_ref, w2_ref, b2_ref,
              y_ref, xg_ref, sem0, sem1):
    i = pl.program_id(0)
    n = pl.num_programs(0)

    def issue(blk, off, sem):
        def body(j, carry):
            t = tok_ref[blk * BM + j]
            pltpu.make_async_copy(x_ref.at[pl.ds(t, 1)],
                                  xg_ref.at[pl.ds(off + j, 1)], sem).start()
            return carry
        lax.fori_loop(0, BM, body, 0)

    def drain(sem):
        pltpu.make_async_copy(x_ref.at[pl.ds(0, BM)],
                              xg_ref.at[pl.ds(0, BM)], sem).wait()

    @pl.when(i == 0)
    def _():
        issue(0, 0, sem0)

    parity = lax.rem(i, 2)

    @pl.when(parity == 0)
    def _():
        drain(sem0)

        @pl.when(i + 1 < n)
        def _():
            issue(i + 1, BM, sem1)

    @pl.when(parity == 1)
    def _():
        drain(sem1)

        @pl.when(i + 1 < n)
        def _():
            issue(i + 1, 0, sem0)

    x = xg_ref[pl.ds(parity * BM, BM), :].astype(jnp.bfloat16)
    h = jnp.dot(x, w1_ref[0].astype(jnp.bfloat16),
                preferred_element_type=jnp.float32) + b1_ref[0]
    h = 0.5 * h * (1.0 + lax.erf(h * 0.7071067811865476))
    y = jnp.dot(h.astype(jnp.bfloat16), w2_ref[0].astype(jnp.bfloat16),
                preferred_element_type=jnp.float32) + b2_ref[0]
    y_ref[...] = y * ws_ref[...]


def _grouped_ffn(x, sorted_tok, w_sorted, block_expert, W1, b1, W2, b2):
    T, D = x.shape
    E, _, F = W1.shape
    npad = w_sorted.shape[0]
    nblk = npad // BM
    grid_spec = pltpu.PrefetchScalarGridSpec(
        num_scalar_prefetch=2,
        grid=(nblk,),
        in_specs=[
            pl.BlockSpec((T, D), lambda i, be, tok: (0, 0)),
            pl.BlockSpec((BM, 1), lambda i, be, tok: (i, 0)),
            pl.BlockSpec((1, D, F), lambda i, be, tok: (be[i], 0, 0)),
            pl.BlockSpec((1, 1, F), lambda i, be, tok: (be[i], 0, 0)),
            pl.BlockSpec((1, F, D), lambda i, be, tok: (be[i], 0, 0)),
            pl.BlockSpec((1, 1, D), lambda i, be, tok: (be[i], 0, 0)),
        ],
        out_specs=pl.BlockSpec((BM, D), lambda i, be, tok: (i, 0)),
        scratch_shapes=[pltpu.VMEM((2 * BM, D), jnp.float32),
                        pltpu.SemaphoreType.DMA,
                        pltpu.SemaphoreType.DMA],
    )
    return pl.pallas_call(
        _ffn_body,
        grid_spec=grid_spec,
        out_shape=jax.ShapeDtypeStruct((npad, D), jnp.float32),
        compiler_params=pltpu.CompilerParams(
            dimension_semantics=("arbitrary",)),
    )(block_expert, sorted_tok, x, w_sorted, W1, b1.reshape(E, 1, F),
      W2, b2.reshape(E, 1, D))


def kernel(tokenHidden, conditionHidden, Wr, br, Wc, W1, b1, W2, b2):
    Bb, Ss, D = tokenHidden.shape
    E = Wr.shape[1]
    T = Bb * Ss
    x = tokenHidden.reshape(T, D)

    topI, nw, aux = _route(x, conditionHidden.reshape(Bb, D), Wr, br, Wc)

    A = T * TOPK
    npad = ((A + E * (BM - 1)) + (NW * 8) - 1) // (NW * 8) * (NW * 8)
    npad = (npad + BM - 1) // BM * BM
    nblk = npad // BM
    onehot = (topI[:, :, None] == jnp.arange(E)[None, None, :])
    M = onehot.any(axis=1).astype(jnp.int32)
    cum = jnp.cumsum(M, axis=0)
    pos = cum - M
    counts = cum[-1]
    nb = (counts + BM - 1) // BM
    cnb = jnp.cumsum(nb)
    pad_off = ((cnb - nb) * BM).astype(jnp.int32)
    slot_te = pad_off[None, :] + pos
    slots = jnp.sum(jnp.where(onehot, slot_te[:, None, :], 0),
                    axis=-1).astype(jnp.int32)
    tok_ids = jnp.arange(T, dtype=jnp.int32)
    slots_all = slots.T.reshape(-1)
    vals = jnp.stack(
        [jnp.concatenate([tok_ids, tok_ids]),
         lax.bitcast_convert_type(nw.T.reshape(-1), jnp.int32)],
        axis=1)
    packed = jnp.zeros((npad, 2), jnp.int32).at[slots_all].set(vals)
    sorted_tok = packed[:, 0]
    sorted_w = lax.bitcast_convert_type(packed[:, 1:2], jnp.float32)
    block_expert = jnp.sum(
        (jnp.arange(nblk, dtype=jnp.int32)[:, None] >= cnb[None, :]),
        axis=-1).astype(jnp.int32)
    block_expert = jnp.minimum(block_expert, E - 1)

    y = _grouped_ffn(x, sorted_tok, sorted_w, block_expert, W1, b1, W2, b2)
    mixed = _sc_combine(y, slots[:, 0], slots[:, 1])

    return mixed.reshape(Bb, Ss, D), aux[0, 0]


# speedup vs baseline: 1.1776x; 1.1776x over previous
import functools

import jax
import jax.numpy as jnp
from jax import lax
from jax.experimental import pallas as pl
from jax.experimental.pallas import tpu as pltpu
from jax.experimental.pallas import tpu_sc as plsc

TOPK = 2
BM = 128
NC = 2
NS = 16
NW = NC * NS


def _router_body(x_ref, cond_ref, wr_ref, br_ref, wc_ref,
                 topi_ref, nw_ref, aux_ref):
    T = x_ref.shape[0]
    E = wr_ref.shape[1]
    x = x_ref[...]
    logits = jnp.dot(x, wr_ref[...], preferred_element_type=jnp.float32)
    cl = jnp.dot(cond_ref[...], wc_ref[...], preferred_element_type=jnp.float32)
    logits = logits + cl + br_ref[...]
    m = jnp.max(logits, axis=-1, keepdims=True)
    p = jnp.exp(logits - m)
    p = p / jnp.sum(p, axis=-1, keepdims=True)

    iota = lax.broadcasted_iota(jnp.int32, (T, E), 1)
    m1 = jnp.max(p, axis=-1, keepdims=True)
    i1 = jnp.min(jnp.where(p >= m1, iota, E), axis=-1, keepdims=True)
    pm = jnp.where(iota == i1, -jnp.inf, p)
    m2 = jnp.max(pm, axis=-1, keepdims=True)
    i2 = jnp.min(jnp.where(pm >= m2, iota, E), axis=-1, keepdims=True)

    s = jnp.clip(m1 + m2, 1e-6, None)
    w1 = m1 / s
    w2 = m2 / s
    topi_ref[...] = jnp.concatenate([i1, i2], axis=1)
    nw_ref[...] = jnp.concatenate([w1, w2], axis=1)

    importance = jnp.sum(p, axis=0) / T
    load = jnp.sum(jnp.where(iota == i1, w1, 0.0)
                   + jnp.where(iota == i2, w2, 0.0), axis=0)
    importance = importance / jnp.clip(jnp.sum(importance), 1e-6, None)
    load = load / jnp.clip(jnp.sum(load), 1e-6, None)
    uni = 1.0 / E
    aux = jnp.mean((importance - uni) ** 2) + jnp.mean((load - uni) ** 2)
    aux_ref[...] = aux.reshape(1, 1)


def _route(x, cond, Wr, br, Wc):
    T = x.shape[0]
    E = Wr.shape[1]
    return pl.pallas_call(
        _router_body,
        out_shape=(
            jax.ShapeDtypeStruct((T, TOPK), jnp.int32),
            jax.ShapeDtypeStruct((T, TOPK), jnp.float32),
            jax.ShapeDtypeStruct((1, 1), jnp.float32),
        ),
    )(x, cond, Wr, br.reshape(1, E), Wc)


def _sc_combine(y, slot0, slot1):
    T = slot0.shape[0]
    D = y.shape[1]
    tpw = T // NW
    mesh = plsc.VectorSubcoreMesh(core_axis_name="c", subcore_axis_name="s")

    @functools.partial(
        pl.kernel, mesh=mesh,
        out_type=jax.ShapeDtypeStruct((T, D), jnp.float32),
        scratch_types=[
            pltpu.VMEM((tpw,), jnp.int32),
            pltpu.VMEM((tpw,), jnp.int32),
            pltpu.VMEM((tpw, D), jnp.float32),
            pltpu.VMEM((tpw, D), jnp.float32),
            pltpu.SemaphoreType.DMA,
        ],
    )
    def k(s0_hbm, s1_hbm, y_hbm, out_hbm, i0_v, i1_v, a_v, b_v, sem):
        wid = lax.axis_index("s") * NC + lax.axis_index("c")
        base = wid * tpw
        pltpu.sync_copy(s0_hbm.at[pl.ds(base, tpw)], i0_v)
        pltpu.sync_copy(s1_hbm.at[pl.ds(base, tpw)], i1_v)
        pltpu.async_copy(y_hbm.at[i0_v], a_v, sem).wait()
        pltpu.async_copy(y_hbm.at[i1_v], b_v, sem).wait()

        def body(r, carry):
            for c in range(D // 16):
                sl = pl.ds(c * 16, 16)
                a_v[r, sl] = a_v[r, sl] + b_v[r, sl]
            return carry

        lax.fori_loop(0, tpw, body, 0)
        pltpu.sync_copy(a_v, out_hbm.at[pl.ds(base, tpw)])

    return k(slot0, slot1, y)


def _ffn_body(be_ref, tok_ref, x_ref, ws_ref, w1_ref, b1_ref, w2_ref, b2_ref,
              y_ref, xg_ref):
    i = pl.program_id(0)

    def gather_row(j, carry):
        t = tok_ref[i * BM + j]
        xg_ref[pl.ds(j, 1), :] = x_ref[pl.ds(t, 1), :]
        return carry

    lax.fori_loop(0, BM, gather_row, 0, unroll=8)
    x = xg_ref[pl.ds(0, BM), :].astype(jnp.bfloat16)
    h = jnp.dot(x, w1_ref[0].astype(jnp.bfloat16),
                preferred_element_type=jnp.float32) + b1_ref[0]
    h = 0.5 * h * (1.0 + lax.erf(h * 0.7071067811865476))
    y = jnp.dot(h.astype(jnp.bfloat16), w2_ref[0].astype(jnp.bfloat16),
                preferred_element_type=jnp.float32) + b2_ref[0]
    y_ref[...] = y * ws_ref[...]


def _grouped_ffn(x, sorted_tok, w_sorted, block_expert, W1, b1, W2, b2):
    T, D = x.shape
    E, _, F = W1.shape
    npad = w_sorted.shape[0]
    nblk = npad // BM
    grid_spec = pltpu.PrefetchScalarGridSpec(
        num_scalar_prefetch=2,
        grid=(nblk,),
        in_specs=[
            pl.BlockSpec((T, D), lambda i, be, tok: (0, 0)),
            pl.BlockSpec((BM, 1), lambda i, be, tok: (i, 0)),
            pl.BlockSpec((1, D, F), lambda i, be, tok: (be[i], 0, 0)),
            pl.BlockSpec((1, 1, F), lambda i, be, tok: (be[i], 0, 0)),
            pl.BlockSpec((1, F, D), lambda i, be, tok: (be[i], 0, 0)),
            pl.BlockSpec((1, 1, D), lambda i, be, tok: (be[i], 0, 0)),
        ],
        out_specs=pl.BlockSpec((BM, D), lambda i, be, tok: (i, 0)),
        scratch_shapes=[pltpu.VMEM((BM, D), jnp.float32)],
    )
    return pl.pallas_call(
        _ffn_body,
        grid_spec=grid_spec,
        out_shape=jax.ShapeDtypeStruct((npad, D), jnp.float32),
        compiler_params=pltpu.CompilerParams(
            dimension_semantics=("arbitrary",)),
    )(block_expert, sorted_tok, x, w_sorted, W1, b1.reshape(E, 1, F),
      W2, b2.reshape(E, 1, D))


def kernel(tokenHidden, conditionHidden, Wr, br, Wc, W1, b1, W2, b2):
    Bb, Ss, D = tokenHidden.shape
    E = Wr.shape[1]
    T = Bb * Ss
    x = tokenHidden.reshape(T, D)

    topI, nw, aux = _route(x, conditionHidden.reshape(Bb, D), Wr, br, Wc)

    A = T * TOPK
    npad = ((A + E * (BM - 1)) + (NW * 8) - 1) // (NW * 8) * (NW * 8)
    npad = (npad + BM - 1) // BM * BM
    nblk = npad // BM
    onehot = (topI[:, :, None] == jnp.arange(E)[None, None, :])
    M = onehot.any(axis=1).astype(jnp.int32)
    cum = jnp.cumsum(M, axis=0)
    pos = cum - M
    counts = cum[-1]
    nb = (counts + BM - 1) // BM
    cnb = jnp.cumsum(nb)
    pad_off = ((cnb - nb) * BM).astype(jnp.int32)
    slot_te = pad_off[None, :] + pos
    slots = jnp.sum(jnp.where(onehot, slot_te[:, None, :], 0),
                    axis=-1).astype(jnp.int32)
    tok_ids = jnp.arange(T, dtype=jnp.int32)
    slots_all = slots.T.reshape(-1)
    vals = jnp.stack(
        [jnp.concatenate([tok_ids, tok_ids]),
         lax.bitcast_convert_type(nw.T.reshape(-1), jnp.int32)],
        axis=1)
    packed = jnp.zeros((npad, 2), jnp.int32).at[slots_all].set(vals)
    sorted_tok = packed[:, 0]
    sorted_w = lax.bitcast_convert_type(packed[:, 1:2], jnp.float32)
    block_expert = jnp.sum(
        (jnp.arange(nblk, dtype=jnp.int32)[:, None] >= cnb[None, :]),
        axis=-1).astype(jnp.int32)
    block_expert = jnp.minimum(block_expert, E - 1)

    y = _grouped_ffn(x, sorted_tok, sorted_w, block_expert, W1, b1, W2, b2)
    mixed = _sc_combine(y, slots[:, 0], slots[:, 1])

    return mixed.reshape(Bb, Ss, D), aux[0, 0]


# speedup vs baseline: 1.1842x; 1.0056x over previous
import functools

import jax
import jax.numpy as jnp
from jax import lax
from jax.experimental import pallas as pl
from jax.experimental.pallas import tpu as pltpu
from jax.experimental.pallas import tpu_sc as plsc

TOPK = 2
BM = 128
NC = 2
NS = 16
NW = NC * NS


def _router_body(x_ref, cond_ref, wr_ref, br_ref, wc_ref,
                 topi_ref, nw_ref, aux_ref):
    T = x_ref.shape[0]
    E = wr_ref.shape[1]
    x = x_ref[...]
    logits = jnp.dot(x, wr_ref[...], preferred_element_type=jnp.float32)
    cl = jnp.dot(cond_ref[...], wc_ref[...], preferred_element_type=jnp.float32)
    logits = logits + cl + br_ref[...]
    m = jnp.max(logits, axis=-1, keepdims=True)
    p = jnp.exp(logits - m)
    p = p / jnp.sum(p, axis=-1, keepdims=True)

    iota = lax.broadcasted_iota(jnp.int32, (T, E), 1)
    m1 = jnp.max(p, axis=-1, keepdims=True)
    i1 = jnp.min(jnp.where(p >= m1, iota, E), axis=-1, keepdims=True)
    pm = jnp.where(iota == i1, -jnp.inf, p)
    m2 = jnp.max(pm, axis=-1, keepdims=True)
    i2 = jnp.min(jnp.where(pm >= m2, iota, E), axis=-1, keepdims=True)

    s = jnp.clip(m1 + m2, 1e-6, None)
    w1 = m1 / s
    w2 = m2 / s
    topi_ref[...] = jnp.concatenate([i1, i2], axis=1)
    nw_ref[...] = jnp.concatenate([w1, w2], axis=1)

    importance = jnp.sum(p, axis=0) / T
    load = jnp.sum(jnp.where(iota == i1, w1, 0.0)
                   + jnp.where(iota == i2, w2, 0.0), axis=0)
    importance = importance / jnp.clip(jnp.sum(importance), 1e-6, None)
    load = load / jnp.clip(jnp.sum(load), 1e-6, None)
    uni = 1.0 / E
    aux = jnp.mean((importance - uni) ** 2) + jnp.mean((load - uni) ** 2)
    aux_ref[...] = aux.reshape(1, 1)


def _route(x, cond, Wr, br, Wc):
    T = x.shape[0]
    E = Wr.shape[1]
    return pl.pallas_call(
        _router_body,
        out_shape=(
            jax.ShapeDtypeStruct((T, TOPK), jnp.int32),
            jax.ShapeDtypeStruct((T, TOPK), jnp.float32),
            jax.ShapeDtypeStruct((1, 1), jnp.float32),
        ),
    )(x, cond, Wr, br.reshape(1, E), Wc)


def _sc_combine(y, slot0, slot1):
    T = slot0.shape[0]
    D = y.shape[1]
    tpw = T // NW
    mesh = plsc.VectorSubcoreMesh(core_axis_name="c", subcore_axis_name="s")

    @functools.partial(
        pl.kernel, mesh=mesh,
        out_type=jax.ShapeDtypeStruct((T, D), jnp.float32),
        scratch_types=[
            pltpu.VMEM((tpw,), jnp.int32),
            pltpu.VMEM((tpw,), jnp.int32),
            pltpu.VMEM((tpw, D), jnp.float32),
            pltpu.VMEM((tpw, D), jnp.float32),
            pltpu.SemaphoreType.DMA,
        ],
    )
    def k(s0_hbm, s1_hbm, y_hbm, out_hbm, i0_v, i1_v, a_v, b_v, sem):
        wid = lax.axis_index("s") * NC + lax.axis_index("c")
        base = wid * tpw
        pltpu.sync_copy(s0_hbm.at[pl.ds(base, tpw)], i0_v)
        pltpu.sync_copy(s1_hbm.at[pl.ds(base, tpw)], i1_v)
        pltpu.async_copy(y_hbm.at[i0_v], a_v, sem).wait()
        pltpu.async_copy(y_hbm.at[i1_v], b_v, sem).wait()

        def body(r, carry):
            for c in range(D // 16):
                sl = pl.ds(c * 16, 16)
                a_v[r, sl] = a_v[r, sl] + b_v[r, sl]
            return carry

        lax.fori_loop(0, tpw, body, 0)
        pltpu.sync_copy(a_v, out_hbm.at[pl.ds(base, tpw)])

    return k(slot0, slot1, y)


def _ffn_body(be_ref, tok_ref, x_ref, ws_ref, w1_ref, b1_ref, w2_ref, b2_ref,
              y_ref, xg_ref):
    i = pl.program_id(0)

    def gather_row(j, carry):
        t = tok_ref[i * BM + j]
        xg_ref[pl.ds(j, 1), :] = x_ref[pl.ds(t, 1), :]
        return carry

    lax.fori_loop(0, BM, gather_row, 0, unroll=16)
    x = xg_ref[pl.ds(0, BM), :].astype(jnp.bfloat16)
    h = jnp.dot(x, w1_ref[0].astype(jnp.bfloat16),
                preferred_element_type=jnp.float32) + b1_ref[0]
    h = 0.5 * h * (1.0 + lax.erf(h * 0.7071067811865476))
    y = jnp.dot(h.astype(jnp.bfloat16), w2_ref[0].astype(jnp.bfloat16),
                preferred_element_type=jnp.float32) + b2_ref[0]
    y_ref[...] = y * ws_ref[...]


def _grouped_ffn(x, sorted_tok, w_sorted, block_expert, W1, b1, W2, b2):
    T, D = x.shape
    E, _, F = W1.shape
    npad = w_sorted.shape[0]
    nblk = npad // BM
    grid_spec = pltpu.PrefetchScalarGridSpec(
        num_scalar_prefetch=2,
        grid=(nblk,),
        in_specs=[
            pl.BlockSpec((T, D), lambda i, be, tok: (0, 0)),
            pl.BlockSpec((BM, 1), lambda i, be, tok: (i, 0)),
            pl.BlockSpec((1, D, F), lambda i, be, tok: (be[i], 0, 0)),
            pl.BlockSpec((1, 1, F), lambda i, be, tok: (be[i], 0, 0)),
            pl.BlockSpec((1, F, D), lambda i, be, tok: (be[i], 0, 0)),
            pl.BlockSpec((1, 1, D), lambda i, be, tok: (be[i], 0, 0)),
        ],
        out_specs=pl.BlockSpec((BM, D), lambda i, be, tok: (i, 0)),
        scratch_shapes=[pltpu.VMEM((BM, D), jnp.float32)],
    )
    return pl.pallas_call(
        _ffn_body,
        grid_spec=grid_spec,
        out_shape=jax.ShapeDtypeStruct((npad, D), jnp.float32),
        compiler_params=pltpu.CompilerParams(
            dimension_semantics=("arbitrary",)),
    )(block_expert, sorted_tok, x, w_sorted, W1, b1.reshape(E, 1, F),
      W2, b2.reshape(E, 1, D))


def kernel(tokenHidden, conditionHidden, Wr, br, Wc, W1, b1, W2, b2):
    Bb, Ss, D = tokenHidden.shape
    E = Wr.shape[1]
    T = Bb * Ss
    x = tokenHidden.reshape(T, D)

    topI, nw, aux = _route(x, conditionHidden.reshape(Bb, D), Wr, br, Wc)

    A = T * TOPK
    npad = ((A + E * (BM - 1)) + (NW * 8) - 1) // (NW * 8) * (NW * 8)
    npad = (npad + BM - 1) // BM * BM
    nblk = npad // BM
    onehot = (topI[:, :, None] == jnp.arange(E)[None, None, :])
    M = onehot.any(axis=1).astype(jnp.int32)
    cum = jnp.cumsum(M, axis=0)
    pos = cum - M
    counts = cum[-1]
    nb = (counts + BM - 1) // BM
    cnb = jnp.cumsum(nb)
    pad_off = ((cnb - nb) * BM).astype(jnp.int32)
    slot_te = pad_off[None, :] + pos
    slots = jnp.sum(jnp.where(onehot, slot_te[:, None, :], 0),
                    axis=-1).astype(jnp.int32)
    tok_ids = jnp.arange(T, dtype=jnp.int32)
    slots_all = slots.T.reshape(-1)
    vals = jnp.stack(
        [jnp.concatenate([tok_ids, tok_ids]),
         lax.bitcast_convert_type(nw.T.reshape(-1), jnp.int32)],
        axis=1)
    packed = jnp.zeros((npad, 2), jnp.int32).at[slots_all].set(vals)
    sorted_tok = packed[:, 0]
    sorted_w = lax.bitcast_convert_type(packed[:, 1:2], jnp.float32)
    block_expert = jnp.sum(
        (jnp.arange(nblk, dtype=jnp.int32)[:, None] >= cnb[None, :]),
        axis=-1).astype(jnp.int32)
    block_expert = jnp.minimum(block_expert, E - 1)

    y = _grouped_ffn(x, sorted_tok, sorted_w, block_expert, W1, b1, W2, b2)
    mixed = _sc_combine(y, slots[:, 0], slots[:, 1])

    return mixed.reshape(Bb, Ss, D), aux[0, 0]


# speedup vs baseline: 1.3287x; 1.1221x over previous
import functools

import jax
import jax.numpy as jnp
from jax import lax
from jax.experimental import pallas as pl
from jax.experimental.pallas import tpu as pltpu
from jax.experimental.pallas import tpu_sc as plsc

TOPK = 2
BM = 128
NC = 2
NS = 16
NW = NC * NS


def _router_body(x_ref, cond_ref, wr_ref, br_ref, wc_ref,
                 topi_ref, nw_ref, aux_ref):
    T = x_ref.shape[0]
    E = wr_ref.shape[1]
    x = x_ref[...]
    logits = jnp.dot(x, wr_ref[...], preferred_element_type=jnp.float32)
    cl = jnp.dot(cond_ref[...], wc_ref[...], preferred_element_type=jnp.float32)
    logits = logits + cl + br_ref[...]
    m = jnp.max(logits, axis=-1, keepdims=True)
    p = jnp.exp(logits - m)
    p = p / jnp.sum(p, axis=-1, keepdims=True)

    iota = lax.broadcasted_iota(jnp.int32, (T, E), 1)
    m1 = jnp.max(p, axis=-1, keepdims=True)
    i1 = jnp.min(jnp.where(p >= m1, iota, E), axis=-1, keepdims=True)
    pm = jnp.where(iota == i1, -jnp.inf, p)
    m2 = jnp.max(pm, axis=-1, keepdims=True)
    i2 = jnp.min(jnp.where(pm >= m2, iota, E), axis=-1, keepdims=True)

    s = jnp.clip(m1 + m2, 1e-6, None)
    w1 = m1 / s
    w2 = m2 / s
    topi_ref[...] = jnp.concatenate([i1, i2], axis=1)
    nw_ref[...] = jnp.concatenate([w1, w2], axis=1)

    importance = jnp.sum(p, axis=0) / T
    load = jnp.sum(jnp.where(iota == i1, w1, 0.0)
                   + jnp.where(iota == i2, w2, 0.0), axis=0)
    importance = importance / jnp.clip(jnp.sum(importance), 1e-6, None)
    load = load / jnp.clip(jnp.sum(load), 1e-6, None)
    uni = 1.0 / E
    aux = jnp.mean((importance - uni) ** 2) + jnp.mean((load - uni) ** 2)
    aux_ref[...] = aux.reshape(1, 1)


def _route(x, cond, Wr, br, Wc):
    T = x.shape[0]
    E = Wr.shape[1]
    return pl.pallas_call(
        _router_body,
        out_shape=(
            jax.ShapeDtypeStruct((T, TOPK), jnp.int32),
            jax.ShapeDtypeStruct((T, TOPK), jnp.float32),
            jax.ShapeDtypeStruct((1, 1), jnp.float32),
        ),
    )(x, cond, Wr, br.reshape(1, E), Wc)


def _sc_combine(y, slot0, slot1):
    T = slot0.shape[0]
    D = y.shape[1]
    tpw = T // NW
    mesh = plsc.VectorSubcoreMesh(core_axis_name="c", subcore_axis_name="s")

    @functools.partial(
        pl.kernel, mesh=mesh,
        out_type=jax.ShapeDtypeStruct((T, D), jnp.float32),
        scratch_types=[
            pltpu.VMEM((tpw,), jnp.int32),
            pltpu.VMEM((tpw,), jnp.int32),
            pltpu.VMEM((tpw, D), jnp.float32),
            pltpu.VMEM((tpw, D), jnp.float32),
            pltpu.SemaphoreType.DMA,
        ],
    )
    def k(s0_hbm, s1_hbm, y_hbm, out_hbm, i0_v, i1_v, a_v, b_v, sem):
        wid = lax.axis_index("s") * NC + lax.axis_index("c")
        base = wid * tpw
        pltpu.sync_copy(s0_hbm.at[pl.ds(base, tpw)], i0_v)
        pltpu.sync_copy(s1_hbm.at[pl.ds(base, tpw)], i1_v)
        pltpu.async_copy(y_hbm.at[i0_v], a_v, sem).wait()
        pltpu.async_copy(y_hbm.at[i1_v], b_v, sem).wait()

        def body(r, carry):
            for c in range(D // 16):
                sl = pl.ds(c * 16, 16)
                a_v[r, sl] = a_v[r, sl] + b_v[r, sl]
            return carry

        lax.fori_loop(0, tpw, body, 0)
        pltpu.sync_copy(a_v, out_hbm.at[pl.ds(base, tpw)])

    return k(slot0, slot1, y)


def _ffn_body(be_ref, tok_ref, first_ref, par_ref, nxt_ref,
              x_ref, ws_ref, w1_hbm, b1_ref, w2_hbm, b2_ref,
              y_ref, xg_ref, w1b_ref, w2b_ref, wsem0, wsem1):
    i = pl.program_id(0)
    wsems = (wsem0, wsem1)

    def issue_w(e, slot):
        pltpu.make_async_copy(w1_hbm.at[e], w1b_ref.at[slot],
                              wsems[slot]).start()
        pltpu.make_async_copy(w2_hbm.at[e], w2b_ref.at[slot],
                              wsems[slot]).start()

    def drain_w(slot):
        pltpu.make_async_copy(w1_hbm.at[0], w1b_ref.at[slot],
                              wsems[slot]).wait()
        pltpu.make_async_copy(w2_hbm.at[0], w2b_ref.at[slot],
                              wsems[slot]).wait()

    @pl.when(i == 0)
    def _():
        issue_w(be_ref[0], 0)

    @pl.when(first_ref[i] == 1)
    def _():
        nxt = nxt_ref[i]
        for p in (0, 1):
            @pl.when(par_ref[i] == p)
            def _():
                drain_w(p)

                @pl.when(nxt >= 0)
                def _():
                    issue_w(nxt, 1 - p)

    def gather_row(j, carry):
        t = tok_ref[i * BM + j]
        xg_ref[pl.ds(j, 1), :] = x_ref[pl.ds(t, 1), :]
        return carry

    lax.fori_loop(0, BM, gather_row, 0, unroll=16)
    par = par_ref[i]
    x = xg_ref[pl.ds(0, BM), :].astype(jnp.bfloat16)
    h = jnp.dot(x, w1b_ref[par].astype(jnp.bfloat16),
                preferred_element_type=jnp.float32) + b1_ref[0]
    h = 0.5 * h * (1.0 + lax.erf(h * 0.7071067811865476))
    y = jnp.dot(h.astype(jnp.bfloat16), w2b_ref[par].astype(jnp.bfloat16),
                preferred_element_type=jnp.float32) + b2_ref[0]
    y_ref[...] = y * ws_ref[...]


def _grouped_ffn(x, sorted_tok, w_sorted, block_expert, first, par, nxt,
                 W1, b1, W2, b2):
    T, D = x.shape
    E, _, F = W1.shape
    npad = w_sorted.shape[0]
    nblk = npad // BM
    grid_spec = pltpu.PrefetchScalarGridSpec(
        num_scalar_prefetch=5,
        grid=(nblk,),
        in_specs=[
            pl.BlockSpec((T, D), lambda i, *_: (0, 0)),
            pl.BlockSpec((BM, 1), lambda i, *_: (i, 0)),
            pl.BlockSpec(memory_space=pl.ANY),
            pl.BlockSpec((1, 1, F), lambda i, be, *_: (be[i], 0, 0)),
            pl.BlockSpec(memory_space=pl.ANY),
            pl.BlockSpec((1, 1, D), lambda i, be, *_: (be[i], 0, 0)),
        ],
        out_specs=pl.BlockSpec((BM, D), lambda i, *_: (i, 0)),
        scratch_shapes=[pltpu.VMEM((BM, D), jnp.float32),
                        pltpu.VMEM((2, D, F), jnp.float32),
                        pltpu.VMEM((2, F, D), jnp.float32),
                        pltpu.SemaphoreType.DMA,
                        pltpu.SemaphoreType.DMA],
    )
    return pl.pallas_call(
        _ffn_body,
        grid_spec=grid_spec,
        out_shape=jax.ShapeDtypeStruct((npad, D), jnp.float32),
        compiler_params=pltpu.CompilerParams(
            dimension_semantics=("arbitrary",)),
    )(block_expert, sorted_tok, first, par, nxt, x, w_sorted,
      W1, b1.reshape(E, 1, F), W2, b2.reshape(E, 1, D))


def kernel(tokenHidden, conditionHidden, Wr, br, Wc, W1, b1, W2, b2):
    Bb, Ss, D = tokenHidden.shape
    E = Wr.shape[1]
    T = Bb * Ss
    x = tokenHidden.reshape(T, D)

    topI, nw, aux = _route(x, conditionHidden.reshape(Bb, D), Wr, br, Wc)

    A = T * TOPK
    npad = ((A + E * (BM - 1)) + (NW * 8) - 1) // (NW * 8) * (NW * 8)
    npad = (npad + BM - 1) // BM * BM
    nblk = npad // BM
    onehot = (topI[:, :, None] == jnp.arange(E)[None, None, :])
    M = onehot.any(axis=1).astype(jnp.int32)
    cum = jnp.cumsum(M, axis=0)
    pos = cum - M
    counts = cum[-1]
    nb = (counts + BM - 1) // BM
    cnb = jnp.cumsum(nb)
    pad_off = ((cnb - nb) * BM).astype(jnp.int32)
    slot_te = pad_off[None, :] + pos
    slots = jnp.sum(jnp.where(onehot, slot_te[:, None, :], 0),
                    axis=-1).astype(jnp.int32)
    tok_ids = jnp.arange(T, dtype=jnp.int32)
    slots_all = slots.T.reshape(-1)
    vals = jnp.stack(
        [jnp.concatenate([tok_ids, tok_ids]),
         lax.bitcast_convert_type(nw.T.reshape(-1), jnp.int32)],
        axis=1)
    packed = jnp.zeros((npad, 2), jnp.int32).at[slots_all].set(vals)
    sorted_tok = packed[:, 0]
    sorted_w = lax.bitcast_convert_type(packed[:, 1:2], jnp.float32)
    ar = jnp.arange(nblk, dtype=jnp.int32)
    block_expert = jnp.sum((ar[:, None] >= cnb[None, :]),
                           axis=-1).astype(jnp.int32)
    block_expert = jnp.minimum(block_expert, E - 1)

    first = jnp.concatenate(
        [jnp.ones((1,), jnp.int32),
         (block_expert[1:] != block_expert[:-1]).astype(jnp.int32)])
    par = ((jnp.cumsum(first) - 1) % 2).astype(jnp.int32)
    idxf = jnp.where(first == 1, ar, nblk)
    shifted = jnp.concatenate([idxf[1:], jnp.full((1,), nblk, jnp.int32)])
    nfi = jnp.flip(lax.cummin(jnp.flip(shifted)))
    nxt = jnp.where(
        nfi < nblk,
        jnp.sum((nfi[:, None] == ar[None, :]) * block_expert[None, :],
                axis=1),
        -1).astype(jnp.int32)

    y = _grouped_ffn(x, sorted_tok, sorted_w, block_expert, first, par, nxt,
                     W1, b1, W2, b2)
    mixed = _sc_combine(y, slots[:, 0], slots[:, 1])

    return mixed.reshape(Bb, Ss, D), aux[0, 0]


# speedup vs baseline: 1.3756x; 1.0353x over previous
import functools

import jax
import jax.numpy as jnp
from jax import lax
from jax.experimental import pallas as pl
from jax.experimental.pallas import tpu as pltpu
from jax.experimental.pallas import tpu_sc as plsc

TOPK = 2
BM = 128
NC = 2
NS = 16
NW = NC * NS


def _router_body(x_ref, cond_ref, wr_ref, br_ref, wc_ref,
                 topi_ref, nw_ref, aux_ref):
    T = x_ref.shape[0]
    E = wr_ref.shape[1]
    x = x_ref[...]
    logits = jnp.dot(x, wr_ref[...], preferred_element_type=jnp.float32)
    cl = jnp.dot(cond_ref[...], wc_ref[...], preferred_element_type=jnp.float32)
    logits = logits + cl + br_ref[...]
    m = jnp.max(logits, axis=-1, keepdims=True)
    p = jnp.exp(logits - m)
    p = p / jnp.sum(p, axis=-1, keepdims=True)

    iota = lax.broadcasted_iota(jnp.int32, (T, E), 1)
    m1 = jnp.max(p, axis=-1, keepdims=True)
    i1 = jnp.min(jnp.where(p >= m1, iota, E), axis=-1, keepdims=True)
    pm = jnp.where(iota == i1, -jnp.inf, p)
    m2 = jnp.max(pm, axis=-1, keepdims=True)
    i2 = jnp.min(jnp.where(pm >= m2, iota, E), axis=-1, keepdims=True)

    s = jnp.clip(m1 + m2, 1e-6, None)
    w1 = m1 / s
    w2 = m2 / s
    topi_ref[...] = jnp.concatenate([i1, i2], axis=1)
    nw_ref[...] = jnp.concatenate([w1, w2], axis=1)

    importance = jnp.sum(p, axis=0) / T
    load = jnp.sum(jnp.where(iota == i1, w1, 0.0)
                   + jnp.where(iota == i2, w2, 0.0), axis=0)
    importance = importance / jnp.clip(jnp.sum(importance), 1e-6, None)
    load = load / jnp.clip(jnp.sum(load), 1e-6, None)
    uni = 1.0 / E
    aux = jnp.mean((importance - uni) ** 2) + jnp.mean((load - uni) ** 2)
    aux_ref[...] = aux.reshape(1, 1)


def _route(x, cond, Wr, br, Wc):
    T = x.shape[0]
    E = Wr.shape[1]
    return pl.pallas_call(
        _router_body,
        out_shape=(
            jax.ShapeDtypeStruct((T, TOPK), jnp.int32),
            jax.ShapeDtypeStruct((T, TOPK), jnp.float32),
            jax.ShapeDtypeStruct((1, 1), jnp.float32),
        ),
    )(x, cond, Wr, br.reshape(1, E), Wc)


def _sc_combine(y, slot0, slot1):
    T = slot0.shape[0]
    D = y.shape[1]
    tpw = T // NW
    mesh = plsc.VectorSubcoreMesh(core_axis_name="c", subcore_axis_name="s")

    @functools.partial(
        pl.kernel, mesh=mesh,
        out_type=jax.ShapeDtypeStruct((T, D), jnp.float32),
        scratch_types=[
            pltpu.VMEM((tpw,), jnp.int32),
            pltpu.VMEM((tpw,), jnp.int32),
            pltpu.VMEM((tpw, D), jnp.float32),
            pltpu.VMEM((tpw, D), jnp.float32),
            pltpu.SemaphoreType.DMA,
        ],
    )
    def k(s0_hbm, s1_hbm, y_hbm, out_hbm, i0_v, i1_v, a_v, b_v, sem):
        wid = lax.axis_index("s") * NC + lax.axis_index("c")
        base = wid * tpw
        pltpu.sync_copy(s0_hbm.at[pl.ds(base, tpw)], i0_v)
        pltpu.sync_copy(s1_hbm.at[pl.ds(base, tpw)], i1_v)
        pltpu.async_copy(y_hbm.at[i0_v], a_v, sem).wait()
        pltpu.async_copy(y_hbm.at[i1_v], b_v, sem).wait()

        def body(r, carry):
            for c in range(D // 16):
                sl = pl.ds(c * 16, 16)
                a_v[r, sl] = a_v[r, sl] + b_v[r, sl]
            return carry

        lax.fori_loop(0, tpw, body, 0)
        pltpu.sync_copy(a_v, out_hbm.at[pl.ds(base, tpw)])

    return k(slot0, slot1, y)


def _ffn_body(be_ref, tok_ref, first_ref, par_ref, nxt_ref,
              x_ref, ws_ref, w1_hbm, b1_ref, w2_hbm, b2_ref,
              y_ref, xg_ref, w1b_ref, w2b_ref, wsem0, wsem1):
    i = pl.program_id(0)
    wsems = (wsem0, wsem1)

    def issue_w(e, slot):
        pltpu.make_async_copy(w1_hbm.at[e], w1b_ref.at[slot],
                              wsems[slot]).start()
        pltpu.make_async_copy(w2_hbm.at[e], w2b_ref.at[slot],
                              wsems[slot]).start()

    def drain_w(slot):
        pltpu.make_async_copy(w1_hbm.at[0], w1b_ref.at[slot],
                              wsems[slot]).wait()
        pltpu.make_async_copy(w2_hbm.at[0], w2b_ref.at[slot],
                              wsems[slot]).wait()

    @pl.when(i == 0)
    def _():
        issue_w(be_ref[0], 0)

    @pl.when(first_ref[i] == 1)
    def _():
        nxt = nxt_ref[i]
        for p in (0, 1):
            @pl.when(par_ref[i] == p)
            def _():
                drain_w(p)

                @pl.when(nxt >= 0)
                def _():
                    issue_w(nxt, 1 - p)

    @pl.when(nxt_ref[i] >= -1)
    def _():
        def gather_row(j, carry):
            t = tok_ref[i * BM + j]
            xg_ref[pl.ds(j, 1), :] = x_ref[pl.ds(t, 1), :]
            return carry

        lax.fori_loop(0, BM, gather_row, 0, unroll=16)
        par = par_ref[i]
        x = xg_ref[pl.ds(0, BM), :].astype(jnp.bfloat16)
        h = jnp.dot(x, w1b_ref[par].astype(jnp.bfloat16),
                    preferred_element_type=jnp.float32) + b1_ref[0]
        h = 0.5 * h * (1.0 + lax.erf(h * 0.7071067811865476))
        y = jnp.dot(h.astype(jnp.bfloat16), w2b_ref[par].astype(jnp.bfloat16),
                    preferred_element_type=jnp.float32) + b2_ref[0]
        y_ref[...] = y * ws_ref[...]


def _grouped_ffn(x, sorted_tok, w_sorted, block_expert, first, par, nxt,
                 W1, b1, W2, b2):
    T, D = x.shape
    E, _, F = W1.shape
    npad = w_sorted.shape[0]
    nblk = npad // BM
    grid_spec = pltpu.PrefetchScalarGridSpec(
        num_scalar_prefetch=5,
        grid=(nblk,),
        in_specs=[
            pl.BlockSpec((T, D), lambda i, *_: (0, 0)),
            pl.BlockSpec((BM, 1), lambda i, *_: (i, 0)),
            pl.BlockSpec(memory_space=pl.ANY),
            pl.BlockSpec((1, 1, F), lambda i, be, *_: (be[i], 0, 0)),
            pl.BlockSpec(memory_space=pl.ANY),
            pl.BlockSpec((1, 1, D), lambda i, be, *_: (be[i], 0, 0)),
        ],
        out_specs=pl.BlockSpec((BM, D), lambda i, *_: (i, 0)),
        scratch_shapes=[pltpu.VMEM((BM, D), jnp.float32),
                        pltpu.VMEM((2, D, F), jnp.float32),
                        pltpu.VMEM((2, F, D), jnp.float32),
                        pltpu.SemaphoreType.DMA,
                        pltpu.SemaphoreType.DMA],
    )
    return pl.pallas_call(
        _ffn_body,
        grid_spec=grid_spec,
        out_shape=jax.ShapeDtypeStruct((npad, D), jnp.float32),
        compiler_params=pltpu.CompilerParams(
            dimension_semantics=("arbitrary",)),
    )(block_expert, sorted_tok, first, par, nxt, x, w_sorted,
      W1, b1.reshape(E, 1, F), W2, b2.reshape(E, 1, D))


def kernel(tokenHidden, conditionHidden, Wr, br, Wc, W1, b1, W2, b2):
    Bb, Ss, D = tokenHidden.shape
    E = Wr.shape[1]
    T = Bb * Ss
    x = tokenHidden.reshape(T, D)

    topI, nw, aux = _route(x, conditionHidden.reshape(Bb, D), Wr, br, Wc)

    A = T * TOPK
    npad = ((A + E * (BM - 1)) + (NW * 8) - 1) // (NW * 8) * (NW * 8)
    npad = (npad + BM - 1) // BM * BM
    nblk = npad // BM
    onehot = (topI[:, :, None] == jnp.arange(E)[None, None, :])
    M = onehot.any(axis=1).astype(jnp.int32)
    cum = jnp.cumsum(M, axis=0)
    pos = cum - M
    counts = cum[-1]
    nb = (counts + BM - 1) // BM
    cnb = jnp.cumsum(nb)
    pad_off = ((cnb - nb) * BM).astype(jnp.int32)
    slot_te = pad_off[None, :] + pos
    slots = jnp.sum(jnp.where(onehot, slot_te[:, None, :], 0),
                    axis=-1).astype(jnp.int32)
    tok_ids = jnp.arange(T, dtype=jnp.int32)
    slots_all = slots.T.reshape(-1)
    vals = jnp.stack(
        [jnp.concatenate([tok_ids, tok_ids]),
         lax.bitcast_convert_type(nw.T.reshape(-1), jnp.int32)],
        axis=1)
    packed = jnp.zeros((npad, 2), jnp.int32).at[slots_all].set(vals)
    sorted_tok = packed[:, 0]
    sorted_w = lax.bitcast_convert_type(packed[:, 1:2], jnp.float32)
    ar = jnp.arange(nblk, dtype=jnp.int32)
    block_expert = jnp.sum((ar[:, None] >= cnb[None, :]),
                           axis=-1).astype(jnp.int32)
    block_expert = jnp.minimum(block_expert, E - 1)

    first = jnp.concatenate(
        [jnp.ones((1,), jnp.int32),
         (block_expert[1:] != block_expert[:-1]).astype(jnp.int32)])
    par = ((jnp.cumsum(first) - 1) % 2).astype(jnp.int32)
    idxf = jnp.where(first == 1, ar, nblk)
    shifted = jnp.concatenate([idxf[1:], jnp.full((1,), nblk, jnp.int32)])
    nfi = jnp.flip(lax.cummin(jnp.flip(shifted)))
    nxt = jnp.where(
        nfi < nblk,
        jnp.sum((nfi[:, None] == ar[None, :]) * block_expert[None, :],
                axis=1),
        -1).astype(jnp.int32)
    nxt = jnp.where(ar < cnb[E - 1], nxt, -2).astype(jnp.int32)

    y = _grouped_ffn(x, sorted_tok, sorted_w, block_expert, first, par, nxt,
                     W1, b1, W2, b2)
    mixed = _sc_combine(y, slots[:, 0], slots[:, 1])

    return mixed.reshape(Bb, Ss, D), aux[0, 0]
